# Initial kernel scaffold; baseline (speedup 1.0000x reference)
import jax
import jax.numpy as jnp
from jax.experimental import pallas as pl


def kernel(vectors, x, V, u, m, senders, modulated_params, W1, A1, B1, M1, W2, A2, B2, M2, W3, A3, B3, M3, W4, A4, B4, M4, Wlin):
    raise NotImplementedError("write your pallas kernel here")


# speedup vs baseline: 18.1437x; 18.1437x over previous
import functools
import math

import jax
import jax.numpy as jnp
import numpy as np
from jax import lax
from jax.experimental import pallas as pl
from jax.experimental.pallas import tpu as pltpu
from jax.experimental.pallas import tpu_sc as plsc

N_NODES = 10000
E = 160000
D_X = 128
MUL = 32
MOD_DIM = 16
RANK = 4
ALPHA = 8.0
HID = 128

C = 128
BE = 2000
GRID = E // BE

NC = 2
NT = 16
B = 128
NCHK = E // B
ACC_N = 10240

_SQ3 = math.sqrt(3.0)

_EXP32 = np.zeros((MUL, 3 * MUL), np.float32)
_TILE3 = np.zeros((3, 3 * MUL), np.float32)
_RED3 = np.zeros((3 * MUL, MUL), np.float32)
for _c in range(MUL):
    for _i in range(3):
        _EXP32[_c, 3 * _c + _i] = 1.0
        _TILE3[_i, 3 * _c + _i] = 1.0
        _RED3[3 * _c + _i, _c] = 1.0


def _pre_kernel(x_ref, m_ref, vec_ref, mod_ref, w1_ref, a1_ref, b1_ref,
                m1_ref, exp_ref, til_ref, vals_ref, xm_ref):
    m = m_ref[...]
    xm = x_ref[...] * m
    s1 = jnp.dot(mod_ref[...], m1_ref[...], preferred_element_type=jnp.float32)
    t = jnp.dot(xm, a1_ref[...], preferred_element_type=jnp.float32)
    w = (jnp.dot(xm, w1_ref[...], preferred_element_type=jnp.float32)
         + jnp.dot(t * s1, b1_ref[...], preferred_element_type=jnp.float32)
         * (ALPHA / RANK))
    w = w * (1.0 / math.sqrt(D_X))
    vec = vec_ref[...]
    r = jnp.sqrt(jnp.sum(vec * vec, axis=-1, keepdims=True))
    y = (_SQ3 * vec) / (r + 1e-12)
    val_s = w[:, :MUL] * m
    vv = (jnp.dot(w[:, MUL:], exp_ref[...], preferred_element_type=jnp.float32)
          * jnp.dot(y, til_ref[...], preferred_element_type=jnp.float32)) * m
    vals_ref[:, 0:MUL] = val_s
    vals_ref[:, MUL:C] = vv
    xm_ref[...] = xm


def _post_kernel(xm_ref, sm_ref, nnb_ref, v_ref, u_ref, mod_ref,
                 w2_ref, a2_ref, b2_ref, m2_ref,
                 w3_ref, a3_ref, b3_ref, m3_ref,
                 w4_ref, a4_ref, b4_ref, m4_ref,
                 exp_ref, red_ref, wtop_ref, wbot_ref,
                 xout_ref, vout_ref):
    sm = sm_ref[...]
    inv = 1.0 / (nnb_ref[...] + 1e-5)
    s_w = sm[:, 0:MUL] * inv
    vw = sm[:, MUL:C] * inv
    v = v_ref[...]
    v_s = v[:, :MUL]
    v_v = v[:, MUL:]
    out0a = s_w * v_s
    out0b = jnp.dot(vw * v_v, red_ref[...],
                    preferred_element_type=jnp.float32) * (1.0 / _SQ3)
    sw_e = jnp.dot(s_w, exp_ref[...], preferred_element_type=jnp.float32)
    vs_e = jnp.dot(v_s, exp_ref[...], preferred_element_type=jnp.float32)
    vout = (jnp.dot(sw_e * v_v, wtop_ref[...], preferred_element_type=jnp.float32)
            + jnp.dot(vw * vs_e, wbot_ref[...], preferred_element_type=jnp.float32)
            ) * (1.0 / math.sqrt(2 * MUL))
    mod = mod_ref[...]

    def lora(h, w_r, a_r, b_r, mm_r):
        s = jnp.dot(mod, mm_r[...], preferred_element_type=jnp.float32)
        t = jnp.dot(h, a_r[...], preferred_element_type=jnp.float32) * s
        o = (jnp.dot(h, w_r[...], preferred_element_type=jnp.float32)
             + jnp.dot(t, b_r[...], preferred_element_type=jnp.float32)
             * (ALPHA / RANK))
        return o * (1.0 / math.sqrt(h.shape[-1]))

    xc = jnp.concatenate([xm_ref[...], out0a, out0b], axis=-1)
    h = lora(xc, w2_ref, a2_ref, b2_ref, m2_ref)
    h = h * jax.nn.sigmoid(h)
    h = lora(h, w3_ref, a3_ref, b3_ref, m3_ref)
    h = h * jax.nn.sigmoid(h)
    h = lora(h, w4_ref, a4_ref, b4_ref, m4_ref)
    xout_ref[...] = u_ref[...] * h
    vout_ref[...] = vout


def _seg_body(vals_hbm, m_hbm, snd_hbm, zb2_hbm, zb1_hbm,
              out_hbm, outm_hbm,
              idx1, buf, mbuf, gbuf, gmbuf, acc, accm, sem):
    tile = lax.axis_index("s")
    core = lax.axis_index("c")

    for k in range(ACC_N // NT // B):
        pltpu.sync_copy(zb2_hbm, acc.at[pl.ds(tile * (ACC_N // NT) + k * B, B)])
        pltpu.sync_copy(zb1_hbm, accm.at[pl.ds(tile * (ACC_N // NT) + k * B, B)])
    plsc.subcore_barrier()

    nj = jnp.where(tile < NCHK - (NCHK // NT) * NT, NCHK // NT + 1, NCHK // NT)

    def sbody(j, carry):
        cj = tile + NT * j
        rs = cj * B
        pltpu.sync_copy(snd_hbm.at[cj], idx1)
        pltpu.sync_copy(vals_hbm.at[pl.ds(rs, B)], buf)
        pltpu.sync_copy(m_hbm.at[pl.ds(rs, B)], mbuf)
        pltpu.sync_copy(buf, acc.at[idx1], add=True)
        pltpu.sync_copy(mbuf, accm.at[idx1], add=True)
        return carry

    lax.fori_loop(0, nj, sbody, 0)
    plsc.subcore_barrier()

    w = core * NT + tile
    nw = NC * NT
    nj2 = jnp.where(w < NCHK - (NCHK // nw) * nw, NCHK // nw + 1, NCHK // nw)

    def gbody(j, carry):
        cj = w + nw * j
        rs = cj * B
        pltpu.sync_copy(snd_hbm.at[cj], idx1)
        pltpu.async_copy(acc.at[idx1], gbuf, sem).wait()
        pltpu.async_copy(accm.at[idx1], gmbuf, sem).wait()
        pltpu.sync_copy(gbuf, out_hbm.at[pl.ds(rs, B)])
        pltpu.sync_copy(gmbuf, outm_hbm.at[pl.ds(rs, B)])
        return carry

    lax.fori_loop(0, nj2, gbody, 0)


@functools.cache
def _seg_call():
    return functools.partial(
        pl.kernel,
        out_type=[
            jax.ShapeDtypeStruct((E, C), jnp.float32),
            jax.ShapeDtypeStruct((E,), jnp.float32),
        ],
        mesh=plsc.VectorSubcoreMesh(core_axis_name="c", subcore_axis_name="s",
                                    num_cores=NC, num_subcores=NT),
        scratch_types=[
            pltpu.VMEM((B,), jnp.int32),
            pltpu.VMEM((B, C), jnp.float32),
            pltpu.VMEM((B,), jnp.float32),
            pltpu.VMEM((B, C), jnp.float32),
            pltpu.VMEM((B,), jnp.float32),
            pltpu.VMEM_SHARED((ACC_N, C), jnp.float32),
            pltpu.VMEM_SHARED((ACC_N,), jnp.float32),
            pltpu.SemaphoreType.DMA,
        ],
    )(_seg_body)


def _edge_spec(d):
    return pl.BlockSpec((BE, d), lambda i: (i, 0))


def _full_spec(shape):
    nd = len(shape)
    return pl.BlockSpec(shape, lambda i: (0,) * nd)


def kernel(vectors, x, V, u, m, senders, modulated_params,
           W1, A1, B1, M1, W2, A2, B2, M2, W3, A3, B3, M3, W4, A4, B4, M4,
           Wlin):
    m2 = m[:, None]
    u2 = u[:, None]
    exp32 = jnp.asarray(_EXP32)
    til3 = jnp.asarray(_TILE3)
    red3 = jnp.asarray(_RED3)
    eye3 = jnp.eye(3, dtype=jnp.float32)
    wtop = jnp.kron(Wlin[:MUL, :], eye3)
    wbot = jnp.kron(Wlin[MUL:, :], eye3)

    vals, xm = pl.pallas_call(
        _pre_kernel,
        grid=(GRID,),
        in_specs=[
            _edge_spec(D_X), _edge_spec(1), _edge_spec(3), _edge_spec(MOD_DIM),
            _full_spec(W1.shape), _full_spec(A1.shape), _full_spec(B1.shape),
            _full_spec(M1.shape), _full_spec(exp32.shape),
            _full_spec(til3.shape),
        ],
        out_specs=[_edge_spec(C), _edge_spec(D_X)],
        out_shape=[
            jax.ShapeDtypeStruct((E, C), jnp.float32),
            jax.ShapeDtypeStruct((E, D_X), jnp.float32),
        ],
    )(x, m2, vectors, modulated_params, W1, A1, B1, M1, exp32, til3)

    snd = senders.astype(jnp.int32).reshape(NCHK, B)
    zb2 = jnp.zeros((B, C), jnp.float32)
    zb1 = jnp.zeros((B,), jnp.float32)
    summed, nnb = _seg_call()(vals, m, snd, zb2, zb1)
    nnb2 = nnb[:, None]

    x_out, v_out = pl.pallas_call(
        _post_kernel,
        grid=(GRID,),
        in_specs=[
            _edge_spec(D_X), _edge_spec(C), _edge_spec(1), _edge_spec(4 * MUL),
            _edge_spec(1), _edge_spec(MOD_DIM),
            _full_spec(W2.shape), _full_spec(A2.shape), _full_spec(B2.shape),
            _full_spec(M2.shape),
            _full_spec(W3.shape), _full_spec(A3.shape), _full_spec(B3.shape),
            _full_spec(M3.shape),
            _full_spec(W4.shape), _full_spec(A4.shape), _full_spec(B4.shape),
            _full_spec(M4.shape),
            _full_spec(exp32.shape), _full_spec(red3.shape),
            _full_spec(wtop.shape), _full_spec(wbot.shape),
        ],
        out_specs=[_edge_spec(D_X), _edge_spec(3 * MUL)],
        out_shape=[
            jax.ShapeDtypeStruct((E, D_X), jnp.float32),
            jax.ShapeDtypeStruct((E, 3 * MUL), jnp.float32),
        ],
    )(xm, summed, nnb2, V, u2, modulated_params,
      W2, A2, B2, M2, W3, A3, B3, M3, W4, A4, B4, M4,
      exp32, red3, wtop, wbot)
    return x_out, v_out


# speedup vs baseline: 19.6544x; 1.0833x over previous
import functools
import math

import jax
import jax.numpy as jnp
import numpy as np
from jax import lax
from jax.experimental import pallas as pl
from jax.experimental.pallas import tpu as pltpu
from jax.experimental.pallas import tpu_sc as plsc

N_NODES = 10000
E = 160000
D_X = 128
MUL = 32
MOD_DIM = 16
RANK = 4
ALPHA = 8.0
HID = 128

C = 128
BE = 3200
GRID = E // BE

NC = 2
NT = 16
B = 128
NCHK = E // B
ACC_N = 10240

_SQ3 = math.sqrt(3.0)
_D0 = (((0,), (0,)), ((), ()))

_EXP32 = np.zeros((MUL, 3 * MUL), np.float32)
_TILE3 = np.zeros((3, 3 * MUL), np.float32)
_RED3 = np.zeros((3 * MUL, MUL), np.float32)
for _c in range(MUL):
    for _i in range(3):
        _EXP32[_c, 3 * _c + _i] = 1.0
        _TILE3[_i, 3 * _c + _i] = 1.0
        _RED3[3 * _c + _i, _c] = 1.0
_ONE11 = np.ones((1, 1), np.float32)
_ONE31 = np.ones((3, 1), np.float32)


def _col(row_blk, ones_ref):
    return lax.dot_general(row_blk, ones_ref, _D0,
                           preferred_element_type=jnp.float32)


def _pre_kernel(x_ref, m_ref, vt_ref, md_ref, w1_ref, a1_ref, b1_ref,
                m1_ref, exp_ref, til_ref, one1_ref, one3_ref, vals_ref):
    m = _col(m_ref[...], one1_ref[...])
    xm = x_ref[...] * m
    s1 = lax.dot_general(md_ref[...], m1_ref[...], _D0,
                         preferred_element_type=jnp.float32)
    t = jnp.dot(xm, a1_ref[...], preferred_element_type=jnp.float32)
    w = (jnp.dot(xm, w1_ref[...], preferred_element_type=jnp.float32)
         + jnp.dot(t * s1, b1_ref[...], preferred_element_type=jnp.float32)
         * (ALPHA / RANK))
    w = w * (1.0 / math.sqrt(D_X))
    vt = vt_ref[...]
    rsq = _col(jnp.sum(vt * vt, axis=0, keepdims=True), one1_ref[...])
    scal = (_SQ3 * m) / (jnp.sqrt(rsq) + 1e-12)
    vtil = lax.dot_general(vt, til_ref[...], _D0,
                           preferred_element_type=jnp.float32)
    vv = jnp.dot(w[:, MUL:], exp_ref[...],
                 preferred_element_type=jnp.float32) * vtil * scal
    vals_ref[:, 0:MUL] = w[:, :MUL] * m
    vals_ref[:, MUL:C] = vv


def _post_kernel(x_ref, m_ref, sm_ref, nnb_ref, v_ref, u_ref, md_ref,
                 w2_ref, a2_ref, b2_ref, m2_ref,
                 w3_ref, a3_ref, b3_ref, m3_ref,
                 w4_ref, a4_ref, b4_ref, m4_ref,
                 exp_ref, red_ref, wtop_ref, wbot_ref, one1_ref,
                 xout_ref, vout_ref):
    sm = sm_ref[...]
    inv = 1.0 / (_col(nnb_ref[...], one1_ref[...]) + 1e-5)
    s_w = sm[:, 0:MUL] * inv
    vw = sm[:, MUL:C] * inv
    v = v_ref[...]
    v_s = v[:, :MUL]
    v_v = v[:, MUL:]
    out0a = s_w * v_s
    out0b = jnp.dot(vw * v_v, red_ref[...],
                    preferred_element_type=jnp.float32) * (1.0 / _SQ3)
    sw_e = jnp.dot(s_w, exp_ref[...], preferred_element_type=jnp.float32)
    vs_e = jnp.dot(v_s, exp_ref[...], preferred_element_type=jnp.float32)
    vout = (jnp.dot(sw_e * v_v, wtop_ref[...], preferred_element_type=jnp.float32)
            + jnp.dot(vw * vs_e, wbot_ref[...], preferred_element_type=jnp.float32)
            ) * (1.0 / math.sqrt(2 * MUL))
    md = md_ref[...]

    def lora(h, w_r, a_r, b_r, mm_r):
        s = lax.dot_general(md, mm_r[...], _D0,
                            preferred_element_type=jnp.float32)
        t = jnp.dot(h, a_r[...], preferred_element_type=jnp.float32) * s
        o = (jnp.dot(h, w_r[...], preferred_element_type=jnp.float32)
             + jnp.dot(t, b_r[...], preferred_element_type=jnp.float32)
             * (ALPHA / RANK))
        return o * (1.0 / math.sqrt(h.shape[-1]))

    m = _col(m_ref[...], one1_ref[...])
    xc = jnp.concatenate([x_ref[...] * m, out0a, out0b], axis=-1)
    h = lora(xc, w2_ref, a2_ref, b2_ref, m2_ref)
    h = h * jax.nn.sigmoid(h)
    h = lora(h, w3_ref, a3_ref, b3_ref, m3_ref)
    h = h * jax.nn.sigmoid(h)
    h = lora(h, w4_ref, a4_ref, b4_ref, m4_ref)
    u = _col(u_ref[...], one1_ref[...])
    xout_ref[...] = u * h
    vout_ref[...] = vout


def _seg_body(vals_hbm, m_hbm, snd_hbm, zb2_hbm, zb1_hbm,
              out_hbm, outm_hbm,
              idx1, buf, mbuf, gbuf, acc, accm, sem):
    tile = lax.axis_index("s")
    core = lax.axis_index("c")
    rpt = ACC_N // NT

    for k in range(rpt // B):
        pltpu.sync_copy(zb2_hbm, acc.at[pl.ds(tile * rpt + k * B, B)])
        pltpu.sync_copy(zb1_hbm, accm.at[pl.ds(tile * rpt + k * B, B)])
    plsc.subcore_barrier()

    nj = jnp.where(tile < NCHK - (NCHK // NT) * NT, NCHK // NT + 1, NCHK // NT)

    def sbody(j, carry):
        cj = tile + NT * j
        rs = cj * B
        pltpu.sync_copy(snd_hbm.at[cj], idx1)
        pltpu.sync_copy(vals_hbm.at[pl.ds(rs, B)], buf)
        pltpu.sync_copy(m_hbm.at[pl.ds(rs, B)], mbuf)
        pltpu.sync_copy(buf, acc.at[idx1], add=True)
        pltpu.sync_copy(mbuf, accm.at[idx1], add=True)
        return carry

    lax.fori_loop(0, nj, sbody, 0)
    plsc.subcore_barrier()

    w = core * NT + tile
    nw = NC * NT
    nj2 = jnp.where(w < NCHK - (NCHK // nw) * nw, NCHK // nw + 1, NCHK // nw)

    def gbody(j, carry):
        cj = w + nw * j
        rs = cj * B
        pltpu.sync_copy(snd_hbm.at[cj], idx1)
        pltpu.async_copy(acc.at[idx1], gbuf, sem).wait()
        pltpu.async_copy(accm.at[idx1], mbuf, sem).wait()
        pltpu.sync_copy(gbuf, out_hbm.at[pl.ds(rs, B)])
        pltpu.sync_copy(mbuf, outm_hbm.at[pl.ds(rs, B)])
        return carry

    lax.fori_loop(0, nj2, gbody, 0)


@functools.cache
def _seg_call():
    return functools.partial(
        pl.kernel,
        out_type=[
            jax.ShapeDtypeStruct((E, C), jnp.float32),
            jax.ShapeDtypeStruct((E,), jnp.float32),
        ],
        mesh=plsc.VectorSubcoreMesh(core_axis_name="c", subcore_axis_name="s",
                                    num_cores=NC, num_subcores=NT),
        scratch_types=[
            pltpu.VMEM((B,), jnp.int32),
            pltpu.VMEM((B, C), jnp.float32),
            pltpu.VMEM((B,), jnp.float32),
            pltpu.VMEM((B, C), jnp.float32),
            pltpu.VMEM_SHARED((ACC_N, C), jnp.float32),
            pltpu.VMEM_SHARED((ACC_N,), jnp.float32),
            pltpu.SemaphoreType.DMA,
        ],
    )(_seg_body)


def _edge_spec(d):
    return pl.BlockSpec((BE, d), lambda i: (i, 0))


def _thin_spec(k):
    return pl.BlockSpec((k, BE), lambda i: (0, i))


def _full_spec(shape):
    nd = len(shape)
    return pl.BlockSpec(shape, lambda i: (0,) * nd)


def kernel(vectors, x, V, u, m, senders, modulated_params,
           W1, A1, B1, M1, W2, A2, B2, M2, W3, A3, B3, M3, W4, A4, B4, M4,
           Wlin):
    m1r = m[None, :]
    u1r = u[None, :]
    vec_t = vectors.T
    mod_t = modulated_params.T
    exp32 = jnp.asarray(_EXP32)
    til3 = jnp.asarray(_TILE3)
    red3 = jnp.asarray(_RED3)
    one11 = jnp.asarray(_ONE11)
    one31 = jnp.asarray(_ONE31)
    eye3 = jnp.eye(3, dtype=jnp.float32)
    wtop = jnp.kron(Wlin[:MUL, :], eye3)
    wbot = jnp.kron(Wlin[MUL:, :], eye3)

    vals = pl.pallas_call(
        _pre_kernel,
        grid=(GRID,),
        in_specs=[
            _edge_spec(D_X), _thin_spec(1), _thin_spec(3), _thin_spec(MOD_DIM),
            _full_spec(W1.shape), _full_spec(A1.shape), _full_spec(B1.shape),
            _full_spec(M1.shape), _full_spec(exp32.shape),
            _full_spec(til3.shape), _full_spec(one11.shape),
            _full_spec(one31.shape),
        ],
        out_specs=_edge_spec(C),
        out_shape=jax.ShapeDtypeStruct((E, C), jnp.float32),
    )(x, m1r, vec_t, mod_t, W1, A1, B1, M1, exp32, til3, one11, one31)

    snd = senders.astype(jnp.int32).reshape(NCHK, B)
    zb2 = jnp.zeros((B, C), jnp.float32)
    zb1 = jnp.zeros((B,), jnp.float32)
    summed, nnb = _seg_call()(vals, m, snd, zb2, zb1)
    nnb1r = nnb[None, :]

    x_out, v_out = pl.pallas_call(
        _post_kernel,
        grid=(GRID,),
        in_specs=[
            _edge_spec(D_X), _thin_spec(1), _edge_spec(C), _thin_spec(1),
            _edge_spec(4 * MUL), _thin_spec(1), _thin_spec(MOD_DIM),
            _full_spec(W2.shape), _full_spec(A2.shape), _full_spec(B2.shape),
            _full_spec(M2.shape),
            _full_spec(W3.shape), _full_spec(A3.shape), _full_spec(B3.shape),
            _full_spec(M3.shape),
            _full_spec(W4.shape), _full_spec(A4.shape), _full_spec(B4.shape),
            _full_spec(M4.shape),
            _full_spec(exp32.shape), _full_spec(red3.shape),
            _full_spec(wtop.shape), _full_spec(wbot.shape),
            _full_spec(one11.shape),
        ],
        out_specs=[_edge_spec(D_X), _edge_spec(3 * MUL)],
        out_shape=[
            jax.ShapeDtypeStruct((E, D_X), jnp.float32),
            jax.ShapeDtypeStruct((E, 3 * MUL), jnp.float32),
        ],
    )(x, m1r, summed, nnb1r, V, u1r, mod_t,
      W2, A2, B2, M2, W3, A3, B3, M3, W4, A4, B4, M4,
      exp32, red3, wtop, wbot, one11)
    return x_out, v_out


# speedup vs baseline: 24.5330x; 1.2482x over previous
import functools
import math

import jax
import jax.numpy as jnp
import numpy as np
from jax import lax
from jax.experimental import pallas as pl
from jax.experimental.pallas import tpu as pltpu
from jax.experimental.pallas import tpu_sc as plsc

N_NODES = 10000
E = 160000
D_X = 128
MUL = 32
MOD_DIM = 16
RANK = 4
ALPHA = 8.0
HID = 128

C = 128
BE = 3200
GRID = E // BE

NC = 2
NT = 16
B = 128
NCHK = E // B
NPAIR = NCHK // 2
PE = 2 * B
ACC_N = 10240

_SQ3 = math.sqrt(3.0)
_D0 = (((0,), (0,)), ((), ()))

_EXP32 = np.zeros((MUL, 3 * MUL), np.float32)
_TILE3 = np.zeros((3, 3 * MUL), np.float32)
_RED3 = np.zeros((3 * MUL, MUL), np.float32)
for _c in range(MUL):
    for _i in range(3):
        _EXP32[_c, 3 * _c + _i] = 1.0
        _TILE3[_i, 3 * _c + _i] = 1.0
        _RED3[3 * _c + _i, _c] = 1.0
_K42 = np.array([[1, 0], [0, 1], [0, 1], [0, 1]], np.float32)
_EYE3 = np.eye(3, dtype=np.float32)


def _pre_kernel(x_ref, m_ref, vt_ref, md_ref, w1s_ref, w1v_ref, a1_ref,
                b1s_ref, b1v_ref, m1_ref, til_ref, k42_ref, vals_ref):
    vt = vt_ref[...]
    stack = jnp.concatenate([m_ref[...], vt * vt], axis=0)
    cols = lax.dot_general(stack, k42_ref[...], _D0,
                           preferred_element_type=jnp.float32)
    m = cols[:, 0:1]
    scal = (_SQ3 * m) / (jnp.sqrt(cols[:, 1:2]) + 1e-12)
    xm = x_ref[...] * m
    s1 = lax.dot_general(md_ref[...], m1_ref[...], _D0,
                         preferred_element_type=jnp.float32)
    t = jnp.dot(xm, a1_ref[...], preferred_element_type=jnp.float32) * s1
    w_s = (jnp.dot(xm, w1s_ref[...], preferred_element_type=jnp.float32)
           + jnp.dot(t, b1s_ref[...], preferred_element_type=jnp.float32))
    wv = (jnp.dot(xm, w1v_ref[...], preferred_element_type=jnp.float32)
          + jnp.dot(t, b1v_ref[...], preferred_element_type=jnp.float32))
    vtil = lax.dot_general(vt, til_ref[...], _D0,
                           preferred_element_type=jnp.float32)
    vals_ref[:, 0:MUL] = w_s * m
    vals_ref[:, MUL:C] = wv * vtil * scal


def _post_kernel(x_ref, m_ref, u_ref, nnb_ref, sm_ref, v_ref, md_ref,
                 w2a_ref, w2b_ref, w2c_ref, a2a_ref, a2b_ref, a2c_ref,
                 b2_ref, m2_ref,
                 w3_ref, a3_ref, b3_ref, m3_ref,
                 w4_ref, a4_ref, b4_ref, m4_ref,
                 exp_ref, red_ref, wtop_ref, wbot_ref, eye3_ref,
                 xout_ref, vout_ref):
    stack = jnp.concatenate([m_ref[...], u_ref[...], nnb_ref[...]], axis=0)
    cols = lax.dot_general(stack, eye3_ref[...], _D0,
                           preferred_element_type=jnp.float32)
    m = cols[:, 0:1]
    u = cols[:, 1:2]
    inv = 1.0 / (cols[:, 2:3] + 1e-5)
    sm = sm_ref[...]
    s_w = sm[:, 0:MUL] * inv
    vw = sm[:, MUL:C] * inv
    v = v_ref[...]
    v_s = v[:, :MUL]
    v_v = v[:, MUL:]
    out0a = s_w * v_s
    out0b = jnp.dot(vw * v_v, red_ref[...],
                    preferred_element_type=jnp.float32)
    sw_e = jnp.dot(s_w, exp_ref[...], preferred_element_type=jnp.float32)
    vs_e = jnp.dot(v_s, exp_ref[...], preferred_element_type=jnp.float32)
    vout = (jnp.dot(sw_e * v_v, wtop_ref[...], preferred_element_type=jnp.float32)
            + jnp.dot(vw * vs_e, wbot_ref[...], preferred_element_type=jnp.float32))
    md = md_ref[...]
    xm = x_ref[...] * m

    s2 = lax.dot_general(md, m2_ref[...], _D0,
                         preferred_element_type=jnp.float32)
    t2 = (jnp.dot(xm, a2a_ref[...], preferred_element_type=jnp.float32)
          + jnp.dot(out0a, a2b_ref[...], preferred_element_type=jnp.float32)
          + jnp.dot(out0b, a2c_ref[...], preferred_element_type=jnp.float32)
          ) * s2
    h = (jnp.dot(xm, w2a_ref[...], preferred_element_type=jnp.float32)
         + jnp.dot(out0a, w2b_ref[...], preferred_element_type=jnp.float32)
         + jnp.dot(out0b, w2c_ref[...], preferred_element_type=jnp.float32)
         + jnp.dot(t2, b2_ref[...], preferred_element_type=jnp.float32))
    h = h * jax.nn.sigmoid(h)

    def lora(hh, w_r, a_r, b_r, mm_r):
        s = lax.dot_general(md, mm_r[...], _D0,
                            preferred_element_type=jnp.float32)
        t = jnp.dot(hh, a_r[...], preferred_element_type=jnp.float32) * s
        return (jnp.dot(hh, w_r[...], preferred_element_type=jnp.float32)
                + jnp.dot(t, b_r[...], preferred_element_type=jnp.float32))

    h = lora(h, w3_ref, a3_ref, b3_ref, m3_ref)
    h = h * jax.nn.sigmoid(h)
    h = lora(h, w4_ref, a4_ref, b4_ref, m4_ref)
    xout_ref[...] = u * h
    vout_ref[...] = vout


def _seg_body(vals_hbm, m_hbm, snd_hbm, zb2_hbm, zb1_hbm,
              out_hbm, outm_hbm,
              idx2, buf2, mbuf2, acc, accm,
              semi, semv, semm, semg, semw, semwm):
    tile = lax.axis_index("s")
    core = lax.axis_index("c")
    rpt = ACC_N // NT

    for k in range(rpt // B):
        pltpu.sync_copy(zb2_hbm, acc.at[pl.ds(tile * rpt + k * B, B)])
        pltpu.sync_copy(zb1_hbm, accm.at[pl.ds(tile * rpt + k * B, B)])
    plsc.subcore_barrier()

    base_p = NCHK // NT
    remp = NCHK - base_p * NT
    p0 = tile * base_p + jnp.minimum(tile, remp)
    nch = base_p + jnp.where(tile < remp, 1, 0)

    def start_loads(p, b):
        pltpu.async_copy(snd_hbm.at[p], idx2.at[b], semi.at[b])
        pltpu.async_copy(vals_hbm.at[pl.ds(p * B, B)], buf2.at[b], semv.at[b])
        pltpu.async_copy(m_hbm.at[pl.ds(p * B, B)], mbuf2.at[b], semm.at[b])

    start_loads(p0, 0)

    def sbody(g, carry):
        b = g % 2
        pltpu.make_async_copy(snd_hbm.at[0], idx2.at[b], semi.at[b]).wait()
        pltpu.make_async_copy(vals_hbm.at[pl.ds(0, B)], buf2.at[b],
                              semv.at[b]).wait()
        pltpu.make_async_copy(m_hbm.at[pl.ds(0, B)], mbuf2.at[b],
                              semm.at[b]).wait()
        pl.when(g + 1 < nch)(lambda: start_loads(p0 + g + 1, (g + 1) % 2))
        pltpu.sync_copy(buf2.at[b], acc.at[idx2.at[b]], add=True)
        pltpu.sync_copy(mbuf2.at[b], accm.at[idx2.at[b]], add=True)
        return carry

    lax.fori_loop(0, nch, sbody, 0)
    plsc.subcore_barrier()

    w = core * NT + tile
    nw = NC * NT
    base_q = NCHK // nw
    rem = NCHK - base_q * nw
    q0 = w * base_q + jnp.minimum(w, rem)
    nq = base_q + jnp.where(w < rem, 1, 0)

    def gbody(g, carry):
        b = g % 2
        p = q0 + g
        pltpu.sync_copy(snd_hbm.at[p], idx2.at[b])

        def wait_writes():
            pltpu.make_async_copy(buf2.at[b], out_hbm.at[pl.ds(0, B)],
                                  semw.at[b]).wait()
            pltpu.make_async_copy(mbuf2.at[b], outm_hbm.at[pl.ds(0, B)],
                                  semwm.at[b]).wait()

        pl.when(g >= 2)(wait_writes)
        pltpu.async_copy(acc.at[idx2.at[b]], buf2.at[b], semg).wait()
        pltpu.async_copy(accm.at[idx2.at[b]], mbuf2.at[b], semg).wait()
        pltpu.async_copy(buf2.at[b], out_hbm.at[pl.ds(p * B, B)], semw.at[b])
        pltpu.async_copy(mbuf2.at[b], outm_hbm.at[pl.ds(p * B, B)],
                         semwm.at[b])
        return carry

    lax.fori_loop(0, nq, gbody, 0)
    for b in range(2):
        pltpu.make_async_copy(buf2.at[b], out_hbm.at[pl.ds(0, B)],
                              semw.at[b]).wait()
        pltpu.make_async_copy(mbuf2.at[b], outm_hbm.at[pl.ds(0, B)],
                              semwm.at[b]).wait()


@functools.cache
def _seg_call():
    return functools.partial(
        pl.kernel,
        out_type=[
            jax.ShapeDtypeStruct((E, C), jnp.float32),
            jax.ShapeDtypeStruct((E,), jnp.float32),
        ],
        mesh=plsc.VectorSubcoreMesh(core_axis_name="c", subcore_axis_name="s",
                                    num_cores=NC, num_subcores=NT),
        scratch_types=[
            pltpu.VMEM((2, B), jnp.int32),
            pltpu.VMEM((2, B, C), jnp.float32),
            pltpu.VMEM((2, B), jnp.float32),
            pltpu.VMEM_SHARED((ACC_N, C), jnp.float32),
            pltpu.VMEM_SHARED((ACC_N,), jnp.float32),
            pltpu.SemaphoreType.DMA((2,)),
            pltpu.SemaphoreType.DMA((2,)),
            pltpu.SemaphoreType.DMA((2,)),
            pltpu.SemaphoreType.DMA,
            pltpu.SemaphoreType.DMA((2,)),
            pltpu.SemaphoreType.DMA((2,)),
        ],
    )(_seg_body)


def _edge_spec(d):
    return pl.BlockSpec((BE, d), lambda i: (i, 0))


def _thin_spec(k):
    return pl.BlockSpec((k, BE), lambda i: (0, i))


def _full_spec(shape):
    nd = len(shape)
    return pl.BlockSpec(shape, lambda i: (0,) * nd)


def kernel(vectors, x, V, u, m, senders, modulated_params,
           W1, A1, B1, M1, W2, A2, B2, M2, W3, A3, B3, M3, W4, A4, B4, M4,
           Wlin):
    f32 = jnp.float32
    m1r = m[None, :]
    u1r = u[None, :]
    vec_t = vectors.T
    mod_t = modulated_params.T
    exp32 = jnp.asarray(_EXP32)
    til3 = jnp.asarray(_TILE3)
    k42 = jnp.asarray(_K42)
    eye3c = jnp.asarray(_EYE3)

    c1 = 1.0 / math.sqrt(D_X)
    w1s = W1[:, :MUL] * c1
    w1v = (W1[:, MUL:] @ exp32) * c1
    b1s = B1[:, :MUL] * (c1 * ALPHA / RANK)
    b1v = (B1[:, MUL:] @ exp32) * (c1 * ALPHA / RANK)
    c2 = 1.0 / math.sqrt(D_X + 2 * MUL)
    w2 = W2 * c2
    b2 = B2 * (c2 * ALPHA / RANK)
    c3 = 1.0 / math.sqrt(HID)
    w3 = W3 * c3
    b3 = B3 * (c3 * ALPHA / RANK)
    w4 = W4 * c3
    b4 = B4 * (c3 * ALPHA / RANK)
    red3 = jnp.asarray(_RED3) * (1.0 / _SQ3)
    ce = 1.0 / math.sqrt(2 * MUL)
    wtop = jnp.kron(Wlin[:MUL, :], eye3c) * ce
    wbot = jnp.kron(Wlin[MUL:, :], eye3c) * ce

    vals = pl.pallas_call(
        _pre_kernel,
        grid=(GRID,),
        in_specs=[
            _edge_spec(D_X), _thin_spec(1), _thin_spec(3), _thin_spec(MOD_DIM),
            _full_spec(w1s.shape), _full_spec(w1v.shape), _full_spec(A1.shape),
            _full_spec(b1s.shape), _full_spec(b1v.shape), _full_spec(M1.shape),
            _full_spec(til3.shape), _full_spec(k42.shape),
        ],
        out_specs=_edge_spec(C),
        out_shape=jax.ShapeDtypeStruct((E, C), f32),
    )(x, m1r, vec_t, mod_t, w1s, w1v, A1, b1s, b1v, M1, til3, k42)

    snd = senders.astype(jnp.int32).reshape(NCHK, B)
    zb2 = jnp.zeros((B, C), f32)
    zb1 = jnp.zeros((B,), f32)
    summed, nnb = _seg_call()(vals, m, snd, zb2, zb1)
    nnb1r = nnb[None, :]

    x_out, v_out = pl.pallas_call(
        _post_kernel,
        grid=(GRID,),
        in_specs=[
            _edge_spec(D_X), _thin_spec(1), _thin_spec(1), _thin_spec(1),
            _edge_spec(C), _edge_spec(4 * MUL), _thin_spec(MOD_DIM),
            _full_spec((D_X, HID)), _full_spec((MUL, HID)),
            _full_spec((MUL, HID)),
            _full_spec((D_X, RANK)), _full_spec((MUL, RANK)),
            _full_spec((MUL, RANK)),
            _full_spec(b2.shape), _full_spec(M2.shape),
            _full_spec(w3.shape), _full_spec(A3.shape), _full_spec(b3.shape),
            _full_spec(M3.shape),
            _full_spec(w4.shape), _full_spec(A4.shape), _full_spec(b4.shape),
            _full_spec(M4.shape),
            _full_spec(exp32.shape), _full_spec(red3.shape),
            _full_spec(wtop.shape), _full_spec(wbot.shape),
            _full_spec(eye3c.shape),
        ],
        out_specs=[_edge_spec(D_X), _edge_spec(3 * MUL)],
        out_shape=[
            jax.ShapeDtypeStruct((E, D_X), f32),
            jax.ShapeDtypeStruct((E, 3 * MUL), f32),
        ],
    )(x, m1r, u1r, nnb1r, summed, V, mod_t,
      w2[:D_X], w2[D_X:D_X + MUL], w2[D_X + MUL:], A2[:D_X],
      A2[D_X:D_X + MUL], A2[D_X + MUL:], b2, M2,
      w3, A3, b3, M3, w4, A4, b4, M4,
      exp32, red3, wtop, wbot, eye3c)
    return x_out, v_out


# speedup vs baseline: 27.5026x; 1.1210x over previous
import functools
import math

import jax
import jax.numpy as jnp
import numpy as np
from jax import lax
from jax.experimental import pallas as pl
from jax.experimental.pallas import tpu as pltpu
from jax.experimental.pallas import tpu_sc as plsc

N_NODES = 10000
E = 160000
D_X = 128
MUL = 32
MOD_DIM = 16
RANK = 4
ALPHA = 8.0
HID = 128

C = 128
BE = 3200
GRID = E // BE

NC = 2
NT = 16
B = 128
NCHK = E // B
NPAIR = NCHK // 2
PE = 2 * B
ACC_N = 10240

_SQ3 = math.sqrt(3.0)
_D0 = (((0,), (0,)), ((), ()))

_EXP32 = np.zeros((MUL, 3 * MUL), np.float32)
_TILE3 = np.zeros((3, 3 * MUL), np.float32)
_RED3 = np.zeros((3 * MUL, MUL), np.float32)
for _c in range(MUL):
    for _i in range(3):
        _EXP32[_c, 3 * _c + _i] = 1.0
        _TILE3[_i, 3 * _c + _i] = 1.0
        _RED3[3 * _c + _i, _c] = 1.0
_K42 = np.array([[1, 0], [0, 1], [0, 1], [0, 1]], np.float32)
_EYE3 = np.eye(3, dtype=np.float32)


def _pre_kernel(x_ref, m_ref, vt_ref, md_ref, w1f_ref, a1_ref,
                b1f_ref, kp_ref, vals_ref):
    vt = vt_ref[...]
    stack = jnp.concatenate([m_ref[...], vt * vt, vt, md_ref[...]],
                            axis=0)
    cols = lax.dot_general(stack, kp_ref[...], _D0,
                           preferred_element_type=jnp.float32)
    vtil = cols[:, 0:96]
    m = cols[:, 96:97]
    scal = (_SQ3 * m) / (jnp.sqrt(cols[:, 97:98]) + 1e-12)
    s1 = cols[:, 98:102]
    xm = x_ref[...] * m
    t = jnp.dot(xm, a1_ref[...], preferred_element_type=jnp.float32) * s1
    w = (jnp.dot(xm, w1f_ref[...], preferred_element_type=jnp.float32)
         + jnp.dot(t, b1f_ref[...], preferred_element_type=jnp.float32))
    vals_ref[:, 0:MUL] = w[:, 0:MUL] * m
    vals_ref[:, MUL:C] = w[:, MUL:C] * vtil * scal


def _post_kernel(x_ref, m_ref, u_ref, nnb_ref, sm_ref, v_ref, md_ref,
                 w2a_ref, w2b_ref, w2c_ref, a2a_ref, a2b_ref, a2c_ref,
                 b2_ref,
                 w3_ref, a3_ref, b3_ref,
                 w4_ref, a4_ref, b4_ref,
                 p1_ref, redp_ref, wtop_ref, wbot_ref, ks_ref,
                 xout_ref, vout_ref):
    stack = jnp.concatenate([m_ref[...], u_ref[...], nnb_ref[...],
                             md_ref[...]], axis=0)
    cols = lax.dot_general(stack, ks_ref[...], _D0,
                           preferred_element_type=jnp.float32)
    m = cols[:, 0:1]
    u = cols[:, 1:2]
    inv = 1.0 / (cols[:, 2:3] + 1e-5)
    s2 = cols[:, 3:7]
    s3 = cols[:, 7:11]
    s4 = cols[:, 11:15]
    swvw = sm_ref[...] * inv
    v = v_ref[...]
    v_v = v[:, MUL:]
    pv = swvw * v
    out0a = pv[:, :MUL]
    out0b = jnp.dot(pv, redp_ref[...], preferred_element_type=jnp.float32)
    sw_e = jnp.dot(swvw, p1_ref[...], preferred_element_type=jnp.float32)
    vs_e = jnp.dot(v, p1_ref[...], preferred_element_type=jnp.float32)
    vout = (jnp.dot(sw_e * v_v, wtop_ref[...], preferred_element_type=jnp.float32)
            + jnp.dot(swvw[:, MUL:] * vs_e, wbot_ref[...],
                      preferred_element_type=jnp.float32))
    xm = x_ref[...] * m

    t2 = (jnp.dot(xm, a2a_ref[...], preferred_element_type=jnp.float32)
          + jnp.dot(out0a, a2b_ref[...], preferred_element_type=jnp.float32)
          + jnp.dot(out0b, a2c_ref[...], preferred_element_type=jnp.float32)
          ) * s2
    h = (jnp.dot(xm, w2a_ref[...], preferred_element_type=jnp.float32)
         + jnp.dot(out0a, w2b_ref[...], preferred_element_type=jnp.float32)
         + jnp.dot(out0b, w2c_ref[...], preferred_element_type=jnp.float32)
         + jnp.dot(t2, b2_ref[...], preferred_element_type=jnp.float32))
    h = h * jax.nn.sigmoid(h)

    def lora(hh, w_r, a_r, b_r, s):
        t = jnp.dot(hh, a_r[...], preferred_element_type=jnp.float32) * s
        return (jnp.dot(hh, w_r[...], preferred_element_type=jnp.float32)
                + jnp.dot(t, b_r[...], preferred_element_type=jnp.float32))

    h = lora(h, w3_ref, a3_ref, b3_ref, s3)
    h = h * jax.nn.sigmoid(h)
    h = lora(h, w4_ref, a4_ref, b4_ref, s4)
    xout_ref[...] = u * h
    vout_ref[...] = vout


def _seg_body(vals_hbm, m_hbm, snd_hbm, zb2_hbm, zb1_hbm,
              out_hbm, outm_hbm,
              idx2, buf2, mbuf2, acc, accm,
              semi, semv, semm, semg, semw, semwm):
    tile = lax.axis_index("s")
    core = lax.axis_index("c")
    rpt = ACC_N // NT

    for k in range(rpt // B):
        pltpu.sync_copy(zb2_hbm, acc.at[pl.ds(tile * rpt + k * B, B)])
        pltpu.sync_copy(zb1_hbm, accm.at[pl.ds(tile * rpt + k * B, B)])
    plsc.subcore_barrier()

    base_p = NCHK // NT
    remp = NCHK - base_p * NT
    p0 = tile * base_p + jnp.minimum(tile, remp)
    nch = base_p + jnp.where(tile < remp, 1, 0)

    def start_loads(p, b):
        pltpu.async_copy(snd_hbm.at[p], idx2.at[b], semi.at[b])
        pltpu.async_copy(vals_hbm.at[pl.ds(p * B, B)], buf2.at[b], semv.at[b])
        pltpu.async_copy(m_hbm.at[pl.ds(p * B, B)], mbuf2.at[b], semm.at[b])

    start_loads(p0, 0)

    def sbody(g, carry):
        b = g % 2
        pltpu.make_async_copy(snd_hbm.at[0], idx2.at[b], semi.at[b]).wait()
        pltpu.make_async_copy(vals_hbm.at[pl.ds(0, B)], buf2.at[b],
                              semv.at[b]).wait()
        pltpu.make_async_copy(m_hbm.at[pl.ds(0, B)], mbuf2.at[b],
                              semm.at[b]).wait()
        pl.when(g + 1 < nch)(lambda: start_loads(p0 + g + 1, (g + 1) % 2))
        pltpu.sync_copy(buf2.at[b], acc.at[idx2.at[b]], add=True)
        pltpu.sync_copy(mbuf2.at[b], accm.at[idx2.at[b]], add=True)
        return carry

    lax.fori_loop(0, nch, sbody, 0)
    plsc.subcore_barrier()

    w = core * NT + tile
    nw = NC * NT
    base_q = NCHK // nw
    rem = NCHK - base_q * nw
    q0 = w * base_q + jnp.minimum(w, rem)
    nq = base_q + jnp.where(w < rem, 1, 0)

    def gbody(g, carry):
        b = g % 2
        p = q0 + g
        pltpu.sync_copy(snd_hbm.at[p], idx2.at[b])

        def wait_writes():
            pltpu.make_async_copy(buf2.at[b], out_hbm.at[pl.ds(0, B)],
                                  semw.at[b]).wait()
            pltpu.make_async_copy(mbuf2.at[b], outm_hbm.at[pl.ds(0, B)],
                                  semwm.at[b]).wait()

        pl.when(g >= 2)(wait_writes)
        pltpu.async_copy(acc.at[idx2.at[b]], buf2.at[b], semg).wait()
        pltpu.async_copy(accm.at[idx2.at[b]], mbuf2.at[b], semg).wait()
        pltpu.async_copy(buf2.at[b], out_hbm.at[pl.ds(p * B, B)], semw.at[b])
        pltpu.async_copy(mbuf2.at[b], outm_hbm.at[pl.ds(p * B, B)],
                         semwm.at[b])
        return carry

    lax.fori_loop(0, nq, gbody, 0)
    for b in range(2):
        pltpu.make_async_copy(buf2.at[b], out_hbm.at[pl.ds(0, B)],
                              semw.at[b]).wait()
        pltpu.make_async_copy(mbuf2.at[b], outm_hbm.at[pl.ds(0, B)],
                              semwm.at[b]).wait()


@functools.cache
def _seg_call():
    return functools.partial(
        pl.kernel,
        out_type=[
            jax.ShapeDtypeStruct((E, C), jnp.float32),
            jax.ShapeDtypeStruct((E,), jnp.float32),
        ],
        mesh=plsc.VectorSubcoreMesh(core_axis_name="c", subcore_axis_name="s",
                                    num_cores=NC, num_subcores=NT),
        scratch_types=[
            pltpu.VMEM((2, B), jnp.int32),
            pltpu.VMEM((2, B, C), jnp.float32),
            pltpu.VMEM((2, B), jnp.float32),
            pltpu.VMEM_SHARED((ACC_N, C), jnp.float32),
            pltpu.VMEM_SHARED((ACC_N,), jnp.float32),
            pltpu.SemaphoreType.DMA((2,)),
            pltpu.SemaphoreType.DMA((2,)),
            pltpu.SemaphoreType.DMA((2,)),
            pltpu.SemaphoreType.DMA,
            pltpu.SemaphoreType.DMA((2,)),
            pltpu.SemaphoreType.DMA((2,)),
        ],
    )(_seg_body)


def _edge_spec(d):
    return pl.BlockSpec((BE, d), lambda i: (i, 0))


def _thin_spec(k):
    return pl.BlockSpec((k, BE), lambda i: (0, i))


def _full_spec(shape):
    nd = len(shape)
    return pl.BlockSpec(shape, lambda i: (0,) * nd)


def kernel(vectors, x, V, u, m, senders, modulated_params,
           W1, A1, B1, M1, W2, A2, B2, M2, W3, A3, B3, M3, W4, A4, B4, M4,
           Wlin):
    f32 = jnp.float32
    m1r = m[None, :]
    u1r = u[None, :]
    vec_t = vectors.T
    mod_t = modulated_params.T
    exp32 = jnp.asarray(_EXP32)

    c1 = 1.0 / math.sqrt(D_X)
    w1f = jnp.concatenate([W1[:, :MUL] * c1, (W1[:, MUL:] @ exp32) * c1],
                          axis=1)
    b1f = jnp.concatenate([B1[:, :MUL], B1[:, MUL:] @ exp32],
                          axis=1) * (c1 * ALPHA / RANK)
    kp_np = np.zeros((7 + MOD_DIM, 102), np.float32)
    kp_np[0, 96] = 1.0
    kp_np[1:4, 97] = 1.0
    kp_np[4:7, 0:96] = _TILE3
    kp = jnp.asarray(kp_np).at[7:, 98:102].set(M1)
    c2 = 1.0 / math.sqrt(D_X + 2 * MUL)
    w2 = W2 * c2
    b2 = B2 * (c2 * ALPHA / RANK)
    c3 = 1.0 / math.sqrt(HID)
    w3 = W3 * c3
    b3 = B3 * (c3 * ALPHA / RANK)
    w4 = W4 * c3
    b4 = B4 * (c3 * ALPHA / RANK)
    redp_np = np.zeros((C, MUL), np.float32)
    redp_np[MUL:, :] = _RED3 * (1.0 / _SQ3)
    redp = jnp.asarray(redp_np)
    p1_np = np.zeros((C, 3 * MUL), np.float32)
    p1_np[:MUL, :] = _EXP32
    p1 = jnp.asarray(p1_np)
    eye3c = jnp.asarray(_EYE3)
    ce = 1.0 / math.sqrt(2 * MUL)
    wtop = jnp.kron(Wlin[:MUL, :], eye3c) * ce
    wbot = jnp.kron(Wlin[MUL:, :], eye3c) * ce
    ks_np = np.zeros((3 + MOD_DIM, 15), np.float32)
    ks_np[0, 0] = 1.0
    ks_np[1, 1] = 1.0
    ks_np[2, 2] = 1.0
    ks = jnp.asarray(ks_np)
    ks = ks.at[3:, 3:7].set(M2).at[3:, 7:11].set(M3).at[3:, 11:15].set(M4)

    vals = pl.pallas_call(
        _pre_kernel,
        grid=(GRID,),
        in_specs=[
            _edge_spec(D_X), _thin_spec(1), _thin_spec(3), _thin_spec(MOD_DIM),
            _full_spec(w1f.shape), _full_spec(A1.shape),
            _full_spec(b1f.shape), _full_spec(kp.shape),
        ],
        out_specs=_edge_spec(C),
        out_shape=jax.ShapeDtypeStruct((E, C), f32),
    )(x, m1r, vec_t, mod_t, w1f, A1, b1f, kp)

    snd = senders.astype(jnp.int32).reshape(NCHK, B)
    zb2 = jnp.zeros((B, C), f32)
    zb1 = jnp.zeros((B,), f32)
    summed, nnb = _seg_call()(vals, m, snd, zb2, zb1)
    nnb1r = nnb[None, :]

    x_out, v_out = pl.pallas_call(
        _post_kernel,
        grid=(GRID,),
        in_specs=[
            _edge_spec(D_X), _thin_spec(1), _thin_spec(1), _thin_spec(1),
            _edge_spec(C), _edge_spec(4 * MUL), _thin_spec(MOD_DIM),
            _full_spec((D_X, HID)), _full_spec((MUL, HID)),
            _full_spec((MUL, HID)),
            _full_spec((D_X, RANK)), _full_spec((MUL, RANK)),
            _full_spec((MUL, RANK)),
            _full_spec(b2.shape),
            _full_spec(w3.shape), _full_spec(A3.shape), _full_spec(b3.shape),
            _full_spec(w4.shape), _full_spec(A4.shape), _full_spec(b4.shape),
            _full_spec(p1.shape), _full_spec(redp.shape),
            _full_spec(wtop.shape), _full_spec(wbot.shape),
            _full_spec(ks.shape),
        ],
        out_specs=[_edge_spec(D_X), _edge_spec(3 * MUL)],
        out_shape=[
            jax.ShapeDtypeStruct((E, D_X), f32),
            jax.ShapeDtypeStruct((E, 3 * MUL), f32),
        ],
    )(x, m1r, u1r, nnb1r, summed, V, mod_t,
      w2[:D_X], w2[D_X:D_X + MUL], w2[D_X + MUL:], A2[:D_X],
      A2[D_X:D_X + MUL], A2[D_X + MUL:], b2,
      w3, A3, b3, w4, A4, b4,
      p1, redp, wtop, wbot, ks)
    return x_out, v_out


# speedup vs baseline: 29.2316x; 1.0629x over previous
import functools
import math

import jax
import jax.numpy as jnp
import numpy as np
from jax import lax
from jax.experimental import pallas as pl
from jax.experimental.pallas import tpu as pltpu
from jax.experimental.pallas import tpu_sc as plsc

N_NODES = 10000
E = 160000
D_X = 128
MUL = 32
MOD_DIM = 16
RANK = 4
ALPHA = 8.0
HID = 128

C = 128
BE = 6400
GRID = E // BE

NC = 2
NT = 16
B = 128
NCHK = E // B
NPAIR = NCHK // 2
PE = 2 * B
ACC_N = 10240

_SQ3 = math.sqrt(3.0)
_D0 = (((0,), (0,)), ((), ()))

_EXP32 = np.zeros((MUL, 3 * MUL), np.float32)
_TILE3 = np.zeros((3, 3 * MUL), np.float32)
_RED3 = np.zeros((3 * MUL, MUL), np.float32)
for _c in range(MUL):
    for _i in range(3):
        _EXP32[_c, 3 * _c + _i] = 1.0
        _TILE3[_i, 3 * _c + _i] = 1.0
        _RED3[3 * _c + _i, _c] = 1.0
_K42 = np.array([[1, 0], [0, 1], [0, 1], [0, 1]], np.float32)
_EYE3 = np.eye(3, dtype=np.float32)


def _pre_kernel(x_ref, m_ref, vt_ref, md_ref, w1f_ref, a1_ref,
                b1f_ref, kp_ref, vals_ref):
    vt = vt_ref[...]
    stack = jnp.concatenate([m_ref[...], vt * vt, vt, md_ref[...]],
                            axis=0)
    cols = lax.dot_general(stack, kp_ref[...], _D0,
                           preferred_element_type=jnp.float32)
    vtil = cols[:, 0:96]
    m = cols[:, 96:97]
    scal = (_SQ3 * m) / (jnp.sqrt(cols[:, 97:98]) + 1e-12)
    s1 = cols[:, 98:102]
    xm = x_ref[...] * m
    t = jnp.dot(xm, a1_ref[...], preferred_element_type=jnp.float32) * s1
    w = (jnp.dot(xm, w1f_ref[...], preferred_element_type=jnp.float32)
         + jnp.dot(t, b1f_ref[...], preferred_element_type=jnp.float32))
    vals_ref[:, 0:MUL] = w[:, 0:MUL] * m
    vals_ref[:, MUL:C] = w[:, MUL:C] * vtil * scal


def _post_kernel(x_ref, m_ref, u_ref, nnb_ref, sm_ref, v_ref, md_ref,
                 w2a_ref, w2b_ref, a2a_ref, a2b_ref,
                 b2_ref,
                 w3_ref, a3_ref, b3_ref,
                 w4_ref, a4_ref, b4_ref,
                 p1_ref, redp_ref, wtop_ref, wbot_ref, ks_ref,
                 xout_ref, vout_ref):
    stack = jnp.concatenate([m_ref[...], u_ref[...], nnb_ref[...],
                             md_ref[...]], axis=0)
    cols = lax.dot_general(stack, ks_ref[...], _D0,
                           preferred_element_type=jnp.float32)
    m = cols[:, 0:1]
    u = cols[:, 1:2]
    inv = 1.0 / (cols[:, 2:3] + 1e-5)
    s2 = cols[:, 3:7]
    s3 = cols[:, 7:11]
    s4 = cols[:, 11:15]
    swvw = sm_ref[...] * inv
    v = v_ref[...]
    v_v = v[:, MUL:]
    pv = swvw * v
    out0ab = jnp.dot(pv, redp_ref[...], preferred_element_type=jnp.float32)
    sw_e = jnp.dot(swvw, p1_ref[...], preferred_element_type=jnp.float32)
    vs_e = jnp.dot(v, p1_ref[...], preferred_element_type=jnp.float32)
    vout = (jnp.dot(sw_e * v_v, wtop_ref[...], preferred_element_type=jnp.float32)
            + jnp.dot(swvw[:, MUL:] * vs_e, wbot_ref[...],
                      preferred_element_type=jnp.float32))
    xm = x_ref[...] * m

    t2 = (jnp.dot(xm, a2a_ref[...], preferred_element_type=jnp.float32)
          + jnp.dot(out0ab, a2b_ref[...], preferred_element_type=jnp.float32)
          ) * s2
    h = (jnp.dot(xm, w2a_ref[...], preferred_element_type=jnp.float32)
         + jnp.dot(out0ab, w2b_ref[...], preferred_element_type=jnp.float32)
         + jnp.dot(t2, b2_ref[...], preferred_element_type=jnp.float32))
    h = h * jax.nn.sigmoid(h)

    def lora(hh, w_r, a_r, b_r, s):
        t = jnp.dot(hh, a_r[...], preferred_element_type=jnp.float32) * s
        return (jnp.dot(hh, w_r[...], preferred_element_type=jnp.float32)
                + jnp.dot(t, b_r[...], preferred_element_type=jnp.float32))

    h = lora(h, w3_ref, a3_ref, b3_ref, s3)
    h = h * jax.nn.sigmoid(h)
    h = lora(h, w4_ref, a4_ref, b4_ref, s4)
    xout_ref[...] = u * h
    vout_ref[...] = vout


def _seg_body(vals_hbm, m_hbm, snd_hbm, zb2_hbm, zb1_hbm,
              out_hbm, outm_hbm,
              idx2, buf2, mbuf2, acc, accm,
              semi, semv, semm, semg, semw, semwm):
    tile = lax.axis_index("s")
    core = lax.axis_index("c")
    rpt = ACC_N // NT

    for k in range(rpt // B):
        pltpu.sync_copy(zb2_hbm, acc.at[pl.ds(tile * rpt + k * B, B)])
        pltpu.sync_copy(zb1_hbm, accm.at[pl.ds(tile * rpt + k * B, B)])
    plsc.subcore_barrier()

    base_p = NCHK // NT
    remp = NCHK - base_p * NT
    p0 = tile * base_p + jnp.minimum(tile, remp)
    nch = base_p + jnp.where(tile < remp, 1, 0)

    def start_loads(p, b):
        pltpu.async_copy(snd_hbm.at[p], idx2.at[b], semi.at[b])
        pltpu.async_copy(vals_hbm.at[pl.ds(p * B, B)], buf2.at[b], semv.at[b])
        pltpu.async_copy(m_hbm.at[pl.ds(p * B, B)], mbuf2.at[b], semm.at[b])

    start_loads(p0, 0)

    def sbody(g, carry):
        b = g % 2
        pltpu.make_async_copy(snd_hbm.at[0], idx2.at[b], semi.at[b]).wait()
        pltpu.make_async_copy(vals_hbm.at[pl.ds(0, B)], buf2.at[b],
                              semv.at[b]).wait()
        pltpu.make_async_copy(m_hbm.at[pl.ds(0, B)], mbuf2.at[b],
                              semm.at[b]).wait()
        pl.when(g + 1 < nch)(lambda: start_loads(p0 + g + 1, (g + 1) % 2))
        pltpu.sync_copy(buf2.at[b], acc.at[idx2.at[b]], add=True)
        pltpu.sync_copy(mbuf2.at[b], accm.at[idx2.at[b]], add=True)
        return carry

    lax.fori_loop(0, nch, sbody, 0)
    plsc.subcore_barrier()

    w = core * NT + tile
    nw = NC * NT
    base_q = NCHK // nw
    rem = NCHK - base_q * nw
    q0 = w * base_q + jnp.minimum(w, rem)
    nq = base_q + jnp.where(w < rem, 1, 0)

    pltpu.async_copy(snd_hbm.at[q0], idx2.at[0], semi.at[0])

    def gbody(g, carry):
        b = g % 2
        p = q0 + g
        pltpu.make_async_copy(snd_hbm.at[0], idx2.at[b], semi.at[b]).wait()

        def fetch_next():
            pltpu.async_copy(snd_hbm.at[p + 1], idx2.at[(g + 1) % 2],
                             semi.at[(g + 1) % 2])

        pl.when(g + 1 < nq)(fetch_next)

        def wait_writes():
            pltpu.make_async_copy(buf2.at[b], out_hbm.at[pl.ds(0, B)],
                                  semw.at[b]).wait()
            pltpu.make_async_copy(mbuf2.at[b], outm_hbm.at[pl.ds(0, B)],
                                  semwm.at[b]).wait()

        pl.when(g >= 2)(wait_writes)
        pltpu.async_copy(acc.at[idx2.at[b]], buf2.at[b], semg).wait()
        pltpu.async_copy(accm.at[idx2.at[b]], mbuf2.at[b], semg).wait()
        pltpu.async_copy(buf2.at[b], out_hbm.at[pl.ds(p * B, B)], semw.at[b])
        pltpu.async_copy(mbuf2.at[b], outm_hbm.at[pl.ds(p * B, B)],
                         semwm.at[b])
        return carry

    lax.fori_loop(0, nq, gbody, 0)
    for b in range(2):
        pltpu.make_async_copy(buf2.at[b], out_hbm.at[pl.ds(0, B)],
                              semw.at[b]).wait()
        pltpu.make_async_copy(mbuf2.at[b], outm_hbm.at[pl.ds(0, B)],
                              semwm.at[b]).wait()


@functools.cache
def _seg_call():
    return functools.partial(
        pl.kernel,
        out_type=[
            jax.ShapeDtypeStruct((E, C), jnp.float32),
            jax.ShapeDtypeStruct((E,), jnp.float32),
        ],
        mesh=plsc.VectorSubcoreMesh(core_axis_name="c", subcore_axis_name="s",
                                    num_cores=NC, num_subcores=NT),
        scratch_types=[
            pltpu.VMEM((2, B), jnp.int32),
            pltpu.VMEM((2, B, C), jnp.float32),
            pltpu.VMEM((2, B), jnp.float32),
            pltpu.VMEM_SHARED((ACC_N, C), jnp.float32),
            pltpu.VMEM_SHARED((ACC_N,), jnp.float32),
            pltpu.SemaphoreType.DMA((2,)),
            pltpu.SemaphoreType.DMA((2,)),
            pltpu.SemaphoreType.DMA((2,)),
            pltpu.SemaphoreType.DMA,
            pltpu.SemaphoreType.DMA((2,)),
            pltpu.SemaphoreType.DMA((2,)),
        ],
    )(_seg_body)


def _edge_spec(d):
    return pl.BlockSpec((BE, d), lambda i: (i, 0))


def _thin_spec(k):
    return pl.BlockSpec((k, BE), lambda i: (0, i))


def _full_spec(shape):
    nd = len(shape)
    return pl.BlockSpec(shape, lambda i: (0,) * nd)


def kernel(vectors, x, V, u, m, senders, modulated_params,
           W1, A1, B1, M1, W2, A2, B2, M2, W3, A3, B3, M3, W4, A4, B4, M4,
           Wlin):
    f32 = jnp.float32
    m1r = m[None, :]
    u1r = u[None, :]
    vec_t = vectors.T
    mod_t = modulated_params.T
    exp32 = jnp.asarray(_EXP32)

    c1 = 1.0 / math.sqrt(D_X)
    w1f = jnp.concatenate([W1[:, :MUL] * c1, (W1[:, MUL:] @ exp32) * c1],
                          axis=1)
    b1f = jnp.concatenate([B1[:, :MUL], B1[:, MUL:] @ exp32],
                          axis=1) * (c1 * ALPHA / RANK)
    kp_np = np.zeros((7 + MOD_DIM, 102), np.float32)
    kp_np[0, 96] = 1.0
    kp_np[1:4, 97] = 1.0
    kp_np[4:7, 0:96] = _TILE3
    kp = jnp.asarray(kp_np).at[7:, 98:102].set(M1)
    c2 = 1.0 / math.sqrt(D_X + 2 * MUL)
    w2 = W2 * c2
    b2 = B2 * (c2 * ALPHA / RANK)
    c3 = 1.0 / math.sqrt(HID)
    w3 = W3 * c3
    b3 = B3 * (c3 * ALPHA / RANK)
    w4 = W4 * c3
    b4 = B4 * (c3 * ALPHA / RANK)
    redp_np = np.zeros((C, 2 * MUL), np.float32)
    redp_np[:MUL, :MUL] = np.eye(MUL, dtype=np.float32)
    redp_np[MUL:, MUL:] = _RED3 * (1.0 / _SQ3)
    redp = jnp.asarray(redp_np)
    p1_np = np.zeros((C, 3 * MUL), np.float32)
    p1_np[:MUL, :] = _EXP32
    p1 = jnp.asarray(p1_np)
    eye3c = jnp.asarray(_EYE3)
    ce = 1.0 / math.sqrt(2 * MUL)
    wtop = jnp.kron(Wlin[:MUL, :], eye3c) * ce
    wbot = jnp.kron(Wlin[MUL:, :], eye3c) * ce
    ks_np = np.zeros((3 + MOD_DIM, 15), np.float32)
    ks_np[0, 0] = 1.0
    ks_np[1, 1] = 1.0
    ks_np[2, 2] = 1.0
    ks = jnp.asarray(ks_np)
    ks = ks.at[3:, 3:7].set(M2).at[3:, 7:11].set(M3).at[3:, 11:15].set(M4)

    vals = pl.pallas_call(
        _pre_kernel,
        grid=(GRID,),
        in_specs=[
            _edge_spec(D_X), _thin_spec(1), _thin_spec(3), _thin_spec(MOD_DIM),
            _full_spec(w1f.shape), _full_spec(A1.shape),
            _full_spec(b1f.shape), _full_spec(kp.shape),
        ],
        out_specs=_edge_spec(C),
        out_shape=jax.ShapeDtypeStruct((E, C), f32),
    )(x, m1r, vec_t, mod_t, w1f, A1, b1f, kp)

    snd = senders.astype(jnp.int32).reshape(NCHK, B)
    zb2 = jnp.zeros((B, C), f32)
    zb1 = jnp.zeros((B,), f32)
    summed, nnb = _seg_call()(vals, m, snd, zb2, zb1)
    nnb1r = nnb[None, :]

    x_out, v_out = pl.pallas_call(
        _post_kernel,
        grid=(GRID,),
        in_specs=[
            _edge_spec(D_X), _thin_spec(1), _thin_spec(1), _thin_spec(1),
            _edge_spec(C), _edge_spec(4 * MUL), _thin_spec(MOD_DIM),
            _full_spec((D_X, HID)), _full_spec((2 * MUL, HID)),
            _full_spec((D_X, RANK)), _full_spec((2 * MUL, RANK)),
            _full_spec(b2.shape),
            _full_spec(w3.shape), _full_spec(A3.shape), _full_spec(b3.shape),
            _full_spec(w4.shape), _full_spec(A4.shape), _full_spec(b4.shape),
            _full_spec(p1.shape), _full_spec(redp.shape),
            _full_spec(wtop.shape), _full_spec(wbot.shape),
            _full_spec(ks.shape),
        ],
        out_specs=[_edge_spec(D_X), _edge_spec(3 * MUL)],
        out_shape=[
            jax.ShapeDtypeStruct((E, D_X), f32),
            jax.ShapeDtypeStruct((E, 3 * MUL), f32),
        ],
    )(x, m1r, u1r, nnb1r, summed, V, mod_t,
      w2[:D_X], w2[D_X:], A2[:D_X], A2[D_X:], b2,
      w3, A3, b3, w4, A4, b4,
      p1, redp, wtop, wbot, ks)
    return x_out, v_out
